# Initial kernel scaffold; baseline (speedup 1.0000x reference)
import jax
import jax.numpy as jnp
from jax.experimental import pallas as pl


def kernel(x, edge_index, W1, b1, W2, b2, W3, b3, W4, b4, W5, b5, W6, b6):
    raise NotImplementedError("write your pallas kernel here")


# speedup vs baseline: 48.2856x; 48.2856x over previous
import functools

import jax
import jax.numpy as jnp
from jax import lax
from jax.experimental import pallas as pl
from jax.experimental.pallas import tpu as pltpu
from jax.experimental.pallas import tpu_sc as plsc

NC = 2
NS = 16
NW = NC * NS
LANES = 16
CH = 128
K = 4


def _make_sc_scatter(D, npad, nmac):
  seg = npad // NS
  assert seg % CH == 0
  if D == 1:
    tbl_s, rows_s, out_s = (npad,), (K, CH), (NC, npad)
  else:
    tbl_s, rows_s, out_s = (npad, D), (K, CH, D), (NC, npad, D)

  mesh = plsc.VectorSubcoreMesh(core_axis_name="c", subcore_axis_name="s")

  @functools.partial(
      pl.kernel,
      out_type=jax.ShapeDtypeStruct(out_s, jnp.float32),
      mesh=mesh,
      compiler_params=pltpu.CompilerParams(use_tc_tiling_on_sc=False),
      scratch_types=[
          pltpu.VMEM((K, CH), jnp.int32),
          pltpu.VMEM((K, CH), jnp.int32),
          pltpu.VMEM(rows_s, jnp.float32),
          pltpu.VMEM_SHARED(tbl_s, jnp.float32),
          pltpu.VMEM_SHARED(tbl_s, jnp.float32),
          pltpu.SemaphoreType.DMA,
          pltpu.SemaphoreType.DMA,
      ],
  )
  def k(tbl_hbm, src_hbm, dst_hbm, zseg_hbm, out_hbm,
        idx_s, idx_d, rows, tbl_sh, acc_sh, gsem, ssem):
    c = lax.axis_index("c")
    s = lax.axis_index("s")
    w = c * NS + s
    pltpu.sync_copy(tbl_hbm.at[pl.ds(s * seg, seg)], tbl_sh.at[pl.ds(s * seg, seg)])
    pltpu.sync_copy(zseg_hbm, acc_sh.at[pl.ds(s * seg, seg)])
    plsc.subcore_barrier()

    row0 = w * (nmac * K)

    def body(m, carry):
      base = pl.multiple_of(row0 + m * K, K)
      cs = pltpu.async_copy(src_hbm.at[pl.ds(base, K)], idx_s, gsem)
      cd = pltpu.async_copy(dst_hbm.at[pl.ds(base, K)], idx_d, gsem)
      cs.wait()
      cd.wait()
      gh = [pltpu.async_copy(tbl_sh.at[idx_s.at[j]], rows.at[j], gsem)
            for j in range(K)]
      sh = []
      for j in range(K):
        gh[j].wait()
        sh.append(pltpu.async_copy(rows.at[j], acc_sh.at[idx_d.at[j]], ssem,
                                   add=True))
      for h in sh:
        h.wait()
      return carry

    lax.fori_loop(0, nmac, body, 0)
    plsc.subcore_barrier()
    pltpu.sync_copy(acc_sh.at[pl.ds(s * seg, seg)],
                    out_hbm.at[c, pl.ds(s * seg, seg)])

  return k


def _tc_deg(degp_ref, x_ref, dinv_ref, g0_ref):
  deg = degp_ref[0] + degp_ref[1] + 1.0
  dinv = lax.rsqrt(jnp.maximum(deg, 1e-12))
  dinv_ref[...] = dinv
  g0_ref[...] = dinv * x_ref[...]


def _tc_l1(p_ref, g_ref, dinv_ref, w_ref, b_ref, out_ref):
  z = dinv_ref[...] * (p_ref[0] + p_ref[1] + g_ref[...])
  h = jnp.maximum(z[:, None] * w_ref[0][None, :] + b_ref[...][None, :], 0.0)
  out_ref[...] = dinv_ref[...][:, None] * h


def _tc_mid(p_ref, g_ref, dinv_ref, w_ref, b_ref, out_ref):
  dinv = dinv_ref[...][:, None]
  z = dinv * (p_ref[0] + p_ref[1] + g_ref[...])
  h = jnp.dot(z, w_ref[...], preferred_element_type=jnp.float32)
  h = jnp.maximum(h + b_ref[...][None, :], 0.0)
  out_ref[...] = dinv * h


def _tc_fin(p_ref, g_ref, dinv_ref, w_ref, b_ref, out_ref):
  dinv = dinv_ref[...][:, None]
  z = dinv * (p_ref[0] + p_ref[1] + g_ref[...])
  o = jnp.dot(z, w_ref[...], preferred_element_type=jnp.float32)
  o = o + b_ref[...][None, :]
  m = jnp.max(o, axis=1, keepdims=True)
  e = jnp.exp(o - m)
  out_ref[...] = (o - m) - jnp.log(jnp.sum(e, axis=1, keepdims=True))


def _tc(fn, out_shape, *args):
  return pl.pallas_call(fn, out_shape=out_shape)(*args)


def kernel(x, edge_index, W1, b1, W2, b2, W3, b3, W4, b4, W5, b5, W6, b6):
  n = x.shape[0]
  e = edge_index.shape[1]
  f32 = jnp.float32

  npad = ((n + 1 + NS * CH - 1) // (NS * CH)) * (NS * CH)
  per_tile = -(-e // (NW * K * CH)) * K * CH
  nmac = per_tile // (K * CH)
  epad = per_tile * NW
  padn = epad - e

  src = edge_index[0]
  dst = edge_index[1]
  pidx = jnp.arange(padn, dtype=jnp.int32)
  pad_src = (pidx * 7919) % n
  pad_dst = n + pidx % (npad - n)
  srcp = jnp.concatenate([src, pad_src]).reshape(epad // CH, CH)
  dstp = jnp.concatenate([dst, pad_dst]).reshape(epad // CH, CH)

  xf = jnp.concatenate([x[:, 0], jnp.zeros((npad - n,), f32)])
  ones_t = jnp.ones((npad,), f32)
  z1 = jnp.zeros((npad // NS,), f32)
  z32 = jnp.zeros((npad // NS, 32), f32)

  sc1 = _make_sc_scatter(1, npad, nmac)
  sc32 = _make_sc_scatter(32, npad, nmac)
  sds = jax.ShapeDtypeStruct

  degp = sc1(ones_t, srcp, dstp, z1)
  dinv, g0 = _tc(_tc_deg, (sds((npad,), f32), sds((npad,), f32)), degp, xf)
  p0 = sc1(g0, srcp, dstp, z1)
  G = _tc(_tc_l1, sds((npad, 32), f32), p0, g0, dinv, W1, b1)
  for W, b in ((W2, b2), (W3, b3), (W4, b4), (W5, b5)):
    p = sc32(G, srcp, dstp, z32)
    G = _tc(_tc_mid, sds((npad, 32), f32), p, G, dinv, W, b)
  p = sc32(G, srcp, dstp, z32)
  out = _tc(_tc_fin, sds((npad, 2), f32), p, G, dinv, W6, b6)
  return out[:n]


# speedup vs baseline: 63.8625x; 1.3226x over previous
import functools

import jax
import jax.numpy as jnp
from jax import lax
from jax.experimental import pallas as pl
from jax.experimental.pallas import tpu as pltpu
from jax.experimental.pallas import tpu_sc as plsc

NC = 2
NS = 16
NW = NC * NS
LANES = 16
CH = 128
K = 4


def _make_sc_scatter(D, npad, nmac):
  seg = npad // NS
  assert seg % CH == 0
  assert nmac % 2 == 0
  npairs = nmac // 2
  if D == 1:
    tbl_s, rows_s, out_s = (npad,), (2, K, CH), (NC, npad)
  else:
    tbl_s, rows_s, out_s = (npad, D), (2, K, CH, D), (NC, npad, D)

  mesh = plsc.VectorSubcoreMesh(core_axis_name="c", subcore_axis_name="s")

  @functools.partial(
      pl.kernel,
      out_type=jax.ShapeDtypeStruct(out_s, jnp.float32),
      mesh=mesh,
      compiler_params=pltpu.CompilerParams(use_tc_tiling_on_sc=False),
      scratch_types=[
          pltpu.VMEM((4, K, CH), jnp.int32),
          pltpu.VMEM((4, K, CH), jnp.int32),
          pltpu.VMEM(rows_s, jnp.float32),
          pltpu.VMEM_SHARED(tbl_s, jnp.float32),
          pltpu.VMEM_SHARED(tbl_s, jnp.float32),
          pltpu.SemaphoreType.DMA,
          pltpu.SemaphoreType.DMA,
          pltpu.SemaphoreType.DMA,
          pltpu.SemaphoreType.DMA,
          pltpu.SemaphoreType.DMA,
          pltpu.SemaphoreType.DMA,
      ],
  )
  def k(tbl_hbm, src_hbm, dst_hbm, zseg_hbm, out_hbm,
        idx_s, idx_d, rows, tbl_sh, acc_sh,
        isem0, isem1, gsem0, gsem1, ssem0, ssem1):
    c = lax.axis_index("c")
    s = lax.axis_index("s")
    w = c * NS + s
    row0 = w * (nmac * K)
    isem = (isem0, isem1)
    gsem = (gsem0, gsem1)
    ssem = (ssem0, ssem1)

    def issue_idx(m, q, b):
      base = pl.multiple_of(row0 + m * K, K)
      pltpu.async_copy(src_hbm.at[pl.ds(base, K)], idx_s.at[q], isem[b])
      pltpu.async_copy(dst_hbm.at[pl.ds(base, K)], idx_d.at[q], isem[b])

    def wait_idx(q, b):
      pltpu.make_async_copy(src_hbm.at[pl.ds(0, K)], idx_s.at[q], isem[b]).wait()
      pltpu.make_async_copy(src_hbm.at[pl.ds(0, K)], idx_d.at[q], isem[b]).wait()

    issue_idx(0, 0, 0)
    issue_idx(1, 1, 1)
    pltpu.sync_copy(tbl_hbm.at[pl.ds(s * seg, seg)], tbl_sh.at[pl.ds(s * seg, seg)])
    pltpu.sync_copy(zseg_hbm, acc_sh.at[pl.ds(s * seg, seg)])
    plsc.subcore_barrier()

    def body(p, carry):
      for b in (0, 1):
        m = 2 * p + b
        qsel = lax.rem(p, 2) * 2 + b

        @pl.when(p > 0)
        def _():
          for j in range(K):
            pltpu.make_async_copy(rows.at[b, j], acc_sh.at[idx_d.at[0, j]],
                                  ssem[b]).wait()

        wait_idx(qsel, b)
        for j in range(K):
          pltpu.async_copy(tbl_sh.at[idx_s.at[qsel, j]], rows.at[b, j], gsem[b])
        for j in range(K):
          pltpu.make_async_copy(tbl_sh.at[idx_s.at[qsel, j]], rows.at[b, j],
                                gsem[b]).wait()

        @pl.when(p < npairs - 1)
        def _():
          issue_idx(m + 2, lax.rem(p + 1, 2) * 2 + b, b)

        for j in range(K):
          pltpu.async_copy(rows.at[b, j], acc_sh.at[idx_d.at[qsel, j]],
                           ssem[b], add=True)
      return carry

    lax.fori_loop(0, npairs, body, 0)
    for b in (0, 1):
      for j in range(K):
        pltpu.make_async_copy(rows.at[b, j], acc_sh.at[idx_d.at[0, j]],
                              ssem[b]).wait()
    plsc.subcore_barrier()
    pltpu.sync_copy(acc_sh.at[pl.ds(s * seg, seg)],
                    out_hbm.at[c, pl.ds(s * seg, seg)])

  return k


def _tc_deg(degp_ref, x_ref, dinv_ref, g0_ref):
  deg = degp_ref[0] + degp_ref[1] + 1.0
  dinv = lax.rsqrt(jnp.maximum(deg, 1e-12))
  dinv_ref[...] = dinv
  g0_ref[...] = dinv * x_ref[...]


def _tc_l1(p_ref, g_ref, dinv_ref, w_ref, b_ref, out_ref):
  z = dinv_ref[...] * (p_ref[0] + p_ref[1] + g_ref[...])
  h = jnp.maximum(z[:, None] * w_ref[0][None, :] + b_ref[...][None, :], 0.0)
  out_ref[...] = dinv_ref[...][:, None] * h


def _tc_mid(p_ref, g_ref, dinv_ref, w_ref, b_ref, out_ref):
  dinv = dinv_ref[...][:, None]
  z = dinv * (p_ref[0] + p_ref[1] + g_ref[...])
  h = jnp.dot(z, w_ref[...], preferred_element_type=jnp.float32)
  h = jnp.maximum(h + b_ref[...][None, :], 0.0)
  out_ref[...] = dinv * h


def _tc_fin(p_ref, g_ref, dinv_ref, w_ref, b_ref, out_ref):
  dinv = dinv_ref[...][:, None]
  z = dinv * (p_ref[0] + p_ref[1] + g_ref[...])
  o = jnp.dot(z, w_ref[...], preferred_element_type=jnp.float32)
  o = o + b_ref[...][None, :]
  m = jnp.max(o, axis=1, keepdims=True)
  e = jnp.exp(o - m)
  out_ref[...] = (o - m) - jnp.log(jnp.sum(e, axis=1, keepdims=True))


def _tc(fn, out_shape, *args):
  return pl.pallas_call(fn, out_shape=out_shape)(*args)


def kernel(x, edge_index, W1, b1, W2, b2, W3, b3, W4, b4, W5, b5, W6, b6):
  n = x.shape[0]
  e = edge_index.shape[1]
  f32 = jnp.float32

  npad = ((n + 1 + NS * CH - 1) // (NS * CH)) * (NS * CH)
  per_tile = -(-e // (NW * K * CH)) * K * CH
  nmac = per_tile // (K * CH)
  epad = per_tile * NW
  padn = epad - e

  src = edge_index[0]
  dst = edge_index[1]
  pidx = jnp.arange(padn, dtype=jnp.int32)
  pad_src = (pidx * 7919) % n
  pad_dst = n + pidx % (npad - n)
  srcp = jnp.concatenate([src, pad_src]).reshape(epad // CH, CH)
  dstp = jnp.concatenate([dst, pad_dst]).reshape(epad // CH, CH)

  xf = jnp.concatenate([x[:, 0], jnp.zeros((npad - n,), f32)])
  ones_t = jnp.ones((npad,), f32)
  z1 = jnp.zeros((npad // NS,), f32)
  z32 = jnp.zeros((npad // NS, 32), f32)

  sc1 = _make_sc_scatter(1, npad, nmac)
  sc32 = _make_sc_scatter(32, npad, nmac)
  sds = jax.ShapeDtypeStruct

  degp = sc1(ones_t, srcp, dstp, z1)
  dinv, g0 = _tc(_tc_deg, (sds((npad,), f32), sds((npad,), f32)), degp, xf)
  p0 = sc1(g0, srcp, dstp, z1)
  G = _tc(_tc_l1, sds((npad, 32), f32), p0, g0, dinv, W1, b1)
  for W, b in ((W2, b2), (W3, b3), (W4, b4), (W5, b5)):
    p = sc32(G, srcp, dstp, z32)
    G = _tc(_tc_mid, sds((npad, 32), f32), p, G, dinv, W, b)
  p = sc32(G, srcp, dstp, z32)
  out = _tc(_tc_fin, sds((npad, 2), f32), p, G, dinv, W6, b6)
  return out[:n]
